# baseline (speedup 1.0000x reference)
import functools

import jax
import jax.numpy as jnp
from jax import lax
from jax.experimental import pallas as pl
from jax.experimental.pallas import tpu as pltpu
from jax.experimental.pallas import tpu_sc as plsc

_B, _L, _D, _BINS = 64, 2048, 512, 100
_D2 = _D // 2
_N = _B * _L
_S = 2
_NSL = _N // _S
_T = 4096
_BPS = _NSL // _T
_NW = 32
_RPW = _NSL // _NW
_CH = 64
_NCH = _RPW // _CH
_NB = 4
_NWAVE = _NCH // _NB


def _pack_table(gene_table):
    t16 = gene_table.astype(jnp.bfloat16)
    lo = lax.bitcast_convert_type(t16[:, :_D2], jnp.uint16).astype(jnp.uint32)
    hi = lax.bitcast_convert_type(t16[:, _D2:], jnp.uint16).astype(jnp.uint32)
    return lax.bitcast_convert_type(lo | (hi << 16), jnp.float32)


def _sc_gather(table_packed, ids3):
    mesh = plsc.VectorSubcoreMesh(core_axis_name="c", subcore_axis_name="s")

    @functools.partial(
        pl.kernel,
        out_type=jax.ShapeDtypeStruct((_NW, _NCH, _CH, _D2), jnp.float32),
        mesh=mesh,
        scratch_types=[
            pltpu.VMEM((_NCH, _CH), jnp.int32),
        ] + [pltpu.VMEM((_CH, _D2), jnp.float32) for _ in range(_NB)]
          + [pltpu.SemaphoreType.DMA for _ in range(2 * _NB)],
    )
    def gather(table_hbm, idx_hbm, out_hbm, idx_v, *scratch):
        bufs = scratch[:_NB]
        gsems = scratch[_NB:2 * _NB]
        ssems = scratch[2 * _NB:]
        wid = lax.axis_index("s") * 2 + lax.axis_index("c")
        pltpu.sync_copy(idx_hbm.at[wid], idx_v)

        for b in range(_NB):
            pltpu.async_copy(table_hbm.at[idx_v.at[b]], bufs[b], gsems[b])

        def wave(g, carry):
            for b in range(_NB):
                c = g * _NB + b
                pltpu.make_async_copy(
                    table_hbm.at[idx_v.at[c]], bufs[b], gsems[b]).wait()
                pltpu.async_copy(bufs[b], out_hbm.at[wid, c], ssems[b])
            for b in range(_NB):
                c = g * _NB + b
                pltpu.make_async_copy(
                    bufs[b], out_hbm.at[wid, c], ssems[b]).wait()
                pltpu.async_copy(
                    table_hbm.at[idx_v.at[c + _NB]], bufs[b], gsems[b])
            return carry

        lax.fori_loop(0, _NWAVE - 1, wave, 0)

        g_last = _NWAVE - 1
        for b in range(_NB):
            c = g_last * _NB + b
            pltpu.make_async_copy(
                table_hbm.at[idx_v.at[c]], bufs[b], gsems[b]).wait()
            pltpu.async_copy(bufs[b], out_hbm.at[wid, c], ssems[b])
        for b in range(_NB):
            c = g_last * _NB + b
            pltpu.make_async_copy(
                bufs[b], out_hbm.at[wid, c], ssems[b]).wait()

    return gather(table_packed, ids3)


def _dense_core(expr_ref, gene_ref, w1_ref, b1_ref, w2_ref,
                b2_ref, bt_ref, pad_ref, out_ref):
    x = expr_ref[...].reshape(_T, 1)
    v1 = x * w1_ref[...] + b1_ref[...]
    v2 = jnp.where(v1 >= 0, v1, 0.1 * v1)
    v3 = v2 + jnp.dot(v2, w2_ref[...],
                      preferred_element_type=jnp.float32) + b2_ref[...]
    m = jnp.max(v3, axis=-1, keepdims=True)
    e = jnp.exp(v3 - m)
    w = e / jnp.sum(e, axis=-1, keepdims=True)
    expr_emb = jnp.dot(w.astype(jnp.bfloat16),
                       bt_ref[...].astype(jnp.bfloat16),
                       preferred_element_type=jnp.float32)
    pad_vec = pad_ref[...].astype(jnp.bfloat16).astype(jnp.float32)
    sel = x != x
    u = lax.bitcast_convert_type(gene_ref[...], jnp.uint32)
    g_lo = lax.bitcast_convert_type(u << 16, jnp.float32)
    g_hi = lax.bitcast_convert_type(u & jnp.uint32(0xFFFF0000),
                                    jnp.float32)
    gene = jnp.concatenate([g_lo, g_hi], axis=1)
    out_ref[...] = gene + jnp.where(sel, pad_vec, expr_emb)


def _dense_slice(s, prev, xm, gene_s, W1, b1r, W2, b2r,
                 bin_table, pad_table):
    if prev is None:
        def body(*refs):
            _dense_core(*refs)
        extra_specs, extra_args, io_alias = [], [], {}
    else:
        def body(prev_ref, *refs):
            del prev_ref
            _dense_core(*refs)
        extra_specs = [pl.BlockSpec(memory_space=pl.ANY)]
        extra_args = [prev]
        io_alias = {0: 0}

    in_specs = extra_specs + [
        pl.BlockSpec((1, 1, _T), lambda i, s=s: (s * _BPS + i, 0, 0)),
        pl.BlockSpec((_T, _D2), lambda i: (i, 0)),
        pl.BlockSpec((1, _BINS), lambda i: (0, 0)),
        pl.BlockSpec((1, _BINS), lambda i: (0, 0)),
        pl.BlockSpec((_BINS, _BINS), lambda i: (0, 0)),
        pl.BlockSpec((1, _BINS), lambda i: (0, 0)),
        pl.BlockSpec((_BINS, _D), lambda i: (0, 0)),
        pl.BlockSpec((1, _D), lambda i: (0, 0)),
    ]
    return pl.pallas_call(
        body,
        grid=(_BPS,),
        in_specs=in_specs,
        out_specs=pl.BlockSpec((_T, _D), lambda i, s=s: (s * _BPS + i, 0)),
        out_shape=jax.ShapeDtypeStruct((_N, _D), jnp.float32),
        input_output_aliases=io_alias,
    )(*extra_args, xm, gene_s, W1, b1r, W2, b2r,
      bin_table, pad_table)


def kernel(expression, gene_ids, encoder_pad_mask, gene_table,
           W1, b1, W2, b2, bin_table, pad_table):
    ids = gene_ids.astype(jnp.int32).reshape(_S, _NW, _NCH, _CH)
    table_packed = _pack_table(gene_table)
    gene_slices = [
        _sc_gather(table_packed, ids[s]).reshape(_NSL, _D2)
        for s in range(_S)
    ]
    xm = jnp.where(encoder_pad_mask, jnp.float32(jnp.nan),
                   expression).reshape(_N // _T, 1, _T)
    b1r = b1.reshape(1, _BINS)
    b2r = b2.reshape(1, _BINS)
    out = None
    for s in range(_S):
        out = _dense_slice(s, out, xm, gene_slices[s],
                           W1, b1r, W2, b2r, bin_table, pad_table)
    return out.reshape(_B, _L, _D)


# speedup vs baseline: 1.0132x; 1.0132x over previous
import functools

import jax
import jax.numpy as jnp
from jax import lax
from jax.experimental import pallas as pl
from jax.experimental.pallas import tpu as pltpu
from jax.experimental.pallas import tpu_sc as plsc

_B, _L, _D, _BINS = 64, 2048, 512, 100
_D2 = _D // 2
_N = _B * _L
_S = 1
_NSL = _N // _S
_T = 4096
_BPS = _NSL // _T
_NW = 32
_RPW = _NSL // _NW
_CH = 64
_NCH = _RPW // _CH
_NB = 4
_NWAVE = _NCH // _NB


def _pack_table(gene_table):
    t16 = gene_table.astype(jnp.bfloat16)
    lo = lax.bitcast_convert_type(t16[:, :_D2], jnp.uint16).astype(jnp.uint32)
    hi = lax.bitcast_convert_type(t16[:, _D2:], jnp.uint16).astype(jnp.uint32)
    return lax.bitcast_convert_type(lo | (hi << 16), jnp.float32)


def _sc_gather(table_packed, ids3):
    mesh = plsc.VectorSubcoreMesh(core_axis_name="c", subcore_axis_name="s")

    @functools.partial(
        pl.kernel,
        out_type=jax.ShapeDtypeStruct((_NW, _NCH, _CH, _D2), jnp.float32),
        mesh=mesh,
        scratch_types=[
            pltpu.VMEM((_NCH, _CH), jnp.int32),
        ] + [pltpu.VMEM((_CH, _D2), jnp.float32) for _ in range(_NB)]
          + [pltpu.SemaphoreType.DMA for _ in range(2 * _NB)],
    )
    def gather(table_hbm, idx_hbm, out_hbm, idx_v, *scratch):
        bufs = scratch[:_NB]
        gsems = scratch[_NB:2 * _NB]
        ssems = scratch[2 * _NB:]
        wid = lax.axis_index("s") * 2 + lax.axis_index("c")
        pltpu.sync_copy(idx_hbm.at[wid], idx_v)

        for b in range(_NB):
            pltpu.async_copy(table_hbm.at[idx_v.at[b]], bufs[b], gsems[b])

        def wave(g, carry):
            for b in range(_NB):
                c = g * _NB + b
                pltpu.make_async_copy(
                    table_hbm.at[idx_v.at[c]], bufs[b], gsems[b]).wait()
                pltpu.async_copy(bufs[b], out_hbm.at[wid, c], ssems[b])
            for b in range(_NB):
                c = g * _NB + b
                pltpu.make_async_copy(
                    bufs[b], out_hbm.at[wid, c], ssems[b]).wait()
                pltpu.async_copy(
                    table_hbm.at[idx_v.at[c + _NB]], bufs[b], gsems[b])
            return carry

        lax.fori_loop(0, _NWAVE - 1, wave, 0)

        g_last = _NWAVE - 1
        for b in range(_NB):
            c = g_last * _NB + b
            pltpu.make_async_copy(
                table_hbm.at[idx_v.at[c]], bufs[b], gsems[b]).wait()
            pltpu.async_copy(bufs[b], out_hbm.at[wid, c], ssems[b])
        for b in range(_NB):
            c = g_last * _NB + b
            pltpu.make_async_copy(
                bufs[b], out_hbm.at[wid, c], ssems[b]).wait()

    return gather(table_packed, ids3)


def _dense_core(expr_ref, gene_ref, w1_ref, b1_ref, w2_ref,
                b2_ref, bt_ref, pad_ref, out_ref):
    x = expr_ref[...].reshape(_T, 1)
    v1 = x * w1_ref[...] + b1_ref[...]
    v2 = jnp.where(v1 >= 0, v1, 0.1 * v1)
    v3 = v2 + jnp.dot(v2, w2_ref[...],
                      preferred_element_type=jnp.float32) + b2_ref[...]
    m = jnp.max(v3, axis=-1, keepdims=True)
    e = jnp.exp(v3 - m)
    w = e / jnp.sum(e, axis=-1, keepdims=True)
    expr_emb = jnp.dot(w.astype(jnp.bfloat16),
                       bt_ref[...].astype(jnp.bfloat16),
                       preferred_element_type=jnp.float32)
    pad_vec = pad_ref[...].astype(jnp.bfloat16).astype(jnp.float32)
    sel = x != x
    u = lax.bitcast_convert_type(gene_ref[...], jnp.uint32)
    g_lo = lax.bitcast_convert_type(u << 16, jnp.float32)
    g_hi = lax.bitcast_convert_type(u & jnp.uint32(0xFFFF0000),
                                    jnp.float32)
    gene = jnp.concatenate([g_lo, g_hi], axis=1)
    out_ref[...] = gene + jnp.where(sel, pad_vec, expr_emb)


def _dense_slice(s, prev, xm, gene_s, W1, b1r, W2, b2r,
                 bin_table, pad_table):
    if prev is None:
        def body(*refs):
            _dense_core(*refs)
        extra_specs, extra_args, io_alias = [], [], {}
    else:
        def body(prev_ref, *refs):
            del prev_ref
            _dense_core(*refs)
        extra_specs = [pl.BlockSpec(memory_space=pl.ANY)]
        extra_args = [prev]
        io_alias = {0: 0}

    in_specs = extra_specs + [
        pl.BlockSpec((1, 1, _T), lambda i, s=s: (s * _BPS + i, 0, 0)),
        pl.BlockSpec((_T, _D2), lambda i: (i, 0)),
        pl.BlockSpec((1, _BINS), lambda i: (0, 0)),
        pl.BlockSpec((1, _BINS), lambda i: (0, 0)),
        pl.BlockSpec((_BINS, _BINS), lambda i: (0, 0)),
        pl.BlockSpec((1, _BINS), lambda i: (0, 0)),
        pl.BlockSpec((_BINS, _D), lambda i: (0, 0)),
        pl.BlockSpec((1, _D), lambda i: (0, 0)),
    ]
    return pl.pallas_call(
        body,
        grid=(_BPS,),
        in_specs=in_specs,
        out_specs=pl.BlockSpec((_T, _D), lambda i, s=s: (s * _BPS + i, 0)),
        out_shape=jax.ShapeDtypeStruct((_N, _D), jnp.float32),
        input_output_aliases=io_alias,
    )(*extra_args, xm, gene_s, W1, b1r, W2, b2r,
      bin_table, pad_table)


def kernel(expression, gene_ids, encoder_pad_mask, gene_table,
           W1, b1, W2, b2, bin_table, pad_table):
    ids = gene_ids.astype(jnp.int32).reshape(_S, _NW, _NCH, _CH)
    table_packed = _pack_table(gene_table)
    gene_slices = [
        _sc_gather(table_packed, ids[s]).reshape(_NSL, _D2)
        for s in range(_S)
    ]
    xm = jnp.where(encoder_pad_mask, jnp.float32(jnp.nan),
                   expression).reshape(_N // _T, 1, _T)
    b1r = b1.reshape(1, _BINS)
    b2r = b2.reshape(1, _BINS)
    out = None
    for s in range(_S):
        out = _dense_slice(s, out, xm, gene_slices[s],
                           W1, b1r, W2, b2r, bin_table, pad_table)
    return out.reshape(_B, _L, _D)
